# baseline (speedup 1.0000x reference)
import jax
import jax.numpy as jnp
from jax import lax
from jax.experimental import pallas as pl
from jax.experimental.pallas import tpu as pltpu
from jax.experimental.pallas import tpu_sc as plsc

B = 2048
D = 768
E = 16
H = 256

NC = 2
NS = 16
NW = NC * NS
CPT = B // NW
L = 16

BLK = 64
MAXB = B // BLK + E
NPAD = MAXB * BLK
NUSED_LANE = MAXB


GB = B


def _gating_body(x_ref, Wg_ref, bg_ref, bias_ref, logits_ref, idx_ref,
                 idx1_ref):
    g = jnp.dot(x_ref[...], Wg_ref[...], preferred_element_type=jnp.float32)
    g = g + bg_ref[...]
    logits_ref[...] = g
    bsd = g + bias_ref[...]
    m = jnp.max(bsd, axis=1, keepdims=True)
    ids = lax.broadcasted_iota(jnp.int32, (GB, E), 1)
    cand = jnp.where(bsd == m, ids, E)
    am = jnp.min(cand, axis=1, keepdims=True)
    idx_ref[...] = am
    idx1_ref[...] = am.reshape(GB)


def _gating(x, Wg, bg, bias):
    return pl.pallas_call(
        _gating_body,
        grid=(B // GB,),
        in_specs=[
            pl.BlockSpec((GB, D), lambda j: (j, 0)),
            pl.BlockSpec((D, E), lambda j: (0, 0)),
            pl.BlockSpec((1, E), lambda j: (0, 0)),
            pl.BlockSpec((1, E), lambda j: (0, 0)),
        ],
        out_specs=[
            pl.BlockSpec((GB, E), lambda j: (j, 0)),
            pl.BlockSpec((GB, 1), lambda j: (j, 0)),
            pl.BlockSpec((GB,), lambda j: (j,)),
        ],
        out_shape=[
            jax.ShapeDtypeStruct((B, E), jnp.float32),
            jax.ShapeDtypeStruct((B, 1), jnp.int32),
            jax.ShapeDtypeStruct((B,), jnp.int32),
        ],
    )(x, Wg, bg.reshape(1, E), bias)


def _route_body(x_hbm, idx_hbm, xs_hbm, pos_hbm, bexp_hbm,
                xrows, idxall, cnt_ref, base_ref, tot_ref, bc_ref, mark_ref,
                posv, bexp_v, sem_x, sem_sc):
    c = lax.axis_index("c")
    s = lax.axis_index("s")
    wid = s * NC + c
    tbase = wid * CPT

    cp_x = pltpu.async_copy(x_hbm.at[pl.ds(tbase, CPT)], xrows, sem_x)
    pltpu.sync_copy(idx_hbm, idxall)

    zeros16 = jnp.zeros((L,), jnp.int32)
    ones16 = jnp.ones((L,), jnp.int32)
    cnt_ref[...] = zeros16
    base_ref[...] = zeros16
    tot_ref[...] = zeros16

    nvec = B // L
    before = wid * (CPT // L)
    for v in range(nvec):
        ev = idxall[pl.ds(v * L, L)]
        plsc.addupdate_scatter(tot_ref, [ev], ones16)
        in_before = jnp.full((L,), v < before)
        plsc.addupdate_scatter(base_ref, [ev], ones16, mask=in_before)

    pranks = []
    mvs = []
    for g in range(CPT // L):
        mv = idxall[pl.ds(tbase + g * L, L)]
        ofs = plsc.load_gather(cnt_ref, [mv])
        wr = zeros16
        for b in range(E):
            m = mv == b
            cs = plsc.cumsum(m.astype(jnp.int32))
            wr = jnp.where(m, cs - 1, wr)
        pranks.append(ofs + wr)
        mvs.append(mv)
        plsc.addupdate_scatter(cnt_ref, [mv], ones16)

    tot = tot_ref[...]
    nblk = (tot + (BLK - 1)) // BLK
    incl = plsc.cumsum(nblk)
    excl = incl - nblk
    bc_ref[...] = excl * BLK + base_ref[...]

    for g in range(CPT // L):
        pv = plsc.load_gather(bc_ref, [mvs[g]]) + pranks[g]
        posv[pl.ds(g * L, L)] = pv

    pltpu.sync_copy(posv, pos_hbm.at[pl.ds(tbase, CPT)])

    cp_x.wait()
    pltpu.async_copy(xrows, xs_hbm.at[posv], sem_sc).wait()

    @pl.when(wid == 0)
    def _():
        n_used = jnp.sum(nblk)
        lanes = jnp.arange(L, dtype=jnp.int32)
        for ch in range(MAXB // L):
            mark_ref[pl.ds(ch * L, L)] = zeros16
        plsc.store_scatter(mark_ref, [excl], lanes, mask=nblk > 0)
        carry = zeros16
        for ch in range(MAXB // L):
            cm = jnp.maximum(plsc.cummax(mark_ref[pl.ds(ch * L, L)]), carry)
            carry = jnp.full((L,), jnp.max(cm), jnp.int32)
            bexp_v[pl.ds(ch * L, L)] = cm
        bexp_v[pl.ds(MAXB, L)] = jnp.full((L,), n_used, jnp.int32)
        pltpu.sync_copy(bexp_v, bexp_hbm)


def _route(x, idx_flat):
    mesh = plsc.VectorSubcoreMesh(core_axis_name="c", subcore_axis_name="s")
    f = pl.kernel(
        _route_body,
        out_type=[
            jax.ShapeDtypeStruct((NPAD, D), jnp.float32),
            jax.ShapeDtypeStruct((B,), jnp.int32),
            jax.ShapeDtypeStruct((MAXB + L,), jnp.int32),
        ],
        mesh=mesh,
        scratch_types=[
            pltpu.VMEM((CPT, D), jnp.float32),
            pltpu.VMEM((B,), jnp.int32),
            pltpu.VMEM((E,), jnp.int32),
            pltpu.VMEM((E,), jnp.int32),
            pltpu.VMEM((E,), jnp.int32),
            pltpu.VMEM((E,), jnp.int32),
            pltpu.VMEM((MAXB,), jnp.int32),
            pltpu.VMEM((CPT,), jnp.int32),
            pltpu.VMEM((MAXB + L,), jnp.int32),
            pltpu.SemaphoreType.DMA,
            pltpu.SemaphoreType.DMA,
        ],
        compiler_params=pltpu.CompilerParams(needs_layout_passes=False),
    )
    return f(x, idx_flat)


def _mlp_body(be_ref, xs_ref, w1_ref, b1_ref, w2_ref, b2_ref, ys_ref):
    j = pl.program_id(0)

    @pl.when(j < be_ref[NUSED_LANE])
    def _():
        e = be_ref[j]
        h = jnp.dot(xs_ref[...], w1_ref[0], preferred_element_type=jnp.float32)
        h = jnp.maximum(h + b1_ref[pl.ds(e, 1), :], 0.0)
        ys_ref[...] = (
            jnp.dot(h, w2_ref[0], preferred_element_type=jnp.float32)
            + b2_ref[pl.ds(e, 1), :]
        )


def _grouped_mlp(bexp, xs, W1, b1, W2, b2):
    def rowblk(j, be):
        return (jnp.minimum(j, be[NUSED_LANE] - 1), 0)

    grid_spec = pltpu.PrefetchScalarGridSpec(
        num_scalar_prefetch=1,
        grid=(MAXB,),
        in_specs=[
            pl.BlockSpec((BLK, D), rowblk),
            pl.BlockSpec((1, D, H), lambda j, be: (be[j], 0, 0)),
            pl.BlockSpec((E, H), lambda j, be: (0, 0)),
            pl.BlockSpec((1, H, D), lambda j, be: (be[j], 0, 0)),
            pl.BlockSpec((E, D), lambda j, be: (0, 0)),
        ],
        out_specs=pl.BlockSpec((BLK, D), rowblk),
    )
    return pl.pallas_call(
        _mlp_body,
        grid_spec=grid_spec,
        out_shape=jax.ShapeDtypeStruct((NPAD, D), jnp.float32),
    )(bexp, xs, W1, b1, W2, b2)


def _unsort_body(ys_hbm, pos_hbm, out_hbm, posv, rows, sem_g):
    c = lax.axis_index("c")
    s = lax.axis_index("s")
    wid = s * NC + c
    tbase = wid * CPT
    pltpu.sync_copy(pos_hbm.at[pl.ds(tbase, CPT)], posv)
    pltpu.async_copy(ys_hbm.at[posv], rows, sem_g).wait()
    pltpu.sync_copy(rows, out_hbm.at[pl.ds(tbase, CPT)])


def _unsort(ys, pos):
    mesh = plsc.VectorSubcoreMesh(core_axis_name="c", subcore_axis_name="s")
    f = pl.kernel(
        _unsort_body,
        out_type=jax.ShapeDtypeStruct((B, D), jnp.float32),
        mesh=mesh,
        scratch_types=[
            pltpu.VMEM((CPT,), jnp.int32),
            pltpu.VMEM((CPT, D), jnp.float32),
            pltpu.SemaphoreType.DMA,
        ],
        compiler_params=pltpu.CompilerParams(needs_layout_passes=False),
    )
    return f(ys, pos)


def kernel(x, Wg, bg, W1, b1, W2, b2, bias):
    logits, idx, idx1 = _gating(x, Wg, bg, bias)
    xs, pos, bexp = _route(x, idx1)
    ys = _grouped_mlp(bexp, xs, W1, b1, W2, b2)
    out = _unsort(ys, pos)
    return (out, logits, idx)


# speedup vs baseline: 1.1791x; 1.1791x over previous
import jax
import jax.numpy as jnp
from jax import lax
from jax.experimental import pallas as pl
from jax.experimental.pallas import tpu as pltpu
from jax.experimental.pallas import tpu_sc as plsc

B = 2048
D = 768
E = 16
H = 256

NC = 2
NS = 16
NW = NC * NS
CPT = B // NW
L = 16

BLK = 128
MAXB = B // BLK + E
NPAD = MAXB * BLK
NUSED_LANE = 32


GB = B


def _gating_body(x_ref, Wg_ref, bg_ref, bias_ref, logits_ref, idx_ref,
                 idx1_ref):
    g = jnp.dot(x_ref[...], Wg_ref[...], preferred_element_type=jnp.float32)
    g = g + bg_ref[...]
    logits_ref[...] = g
    bsd = g + bias_ref[...]
    m = jnp.max(bsd, axis=1, keepdims=True)
    ids = lax.broadcasted_iota(jnp.int32, (GB, E), 1)
    cand = jnp.where(bsd == m, ids, E)
    am = jnp.min(cand, axis=1, keepdims=True)
    idx_ref[...] = am
    idx1_ref[...] = am.reshape(GB)


def _gating(x, Wg, bg, bias):
    return pl.pallas_call(
        _gating_body,
        grid=(B // GB,),
        in_specs=[
            pl.BlockSpec((GB, D), lambda j: (j, 0)),
            pl.BlockSpec((D, E), lambda j: (0, 0)),
            pl.BlockSpec((1, E), lambda j: (0, 0)),
            pl.BlockSpec((1, E), lambda j: (0, 0)),
        ],
        out_specs=[
            pl.BlockSpec((GB, E), lambda j: (j, 0)),
            pl.BlockSpec((GB, 1), lambda j: (j, 0)),
            pl.BlockSpec((GB,), lambda j: (j,)),
        ],
        out_shape=[
            jax.ShapeDtypeStruct((B, E), jnp.float32),
            jax.ShapeDtypeStruct((B, 1), jnp.int32),
            jax.ShapeDtypeStruct((B,), jnp.int32),
        ],
    )(x, Wg, bg.reshape(1, E), bias)


def _route_body(x_hbm, idx_hbm, xs_hbm, pos_hbm, bexp_hbm,
                xrows, idxall, cnt_ref, base_ref, tot_ref, bc_ref, mark_ref,
                posv, bexp_v, sem_x, sem_sc):
    c = lax.axis_index("c")
    s = lax.axis_index("s")
    wid = s * NC + c
    tbase = wid * CPT

    cp_x = pltpu.async_copy(x_hbm.at[pl.ds(tbase, CPT)], xrows, sem_x)
    pltpu.sync_copy(idx_hbm, idxall)

    zeros16 = jnp.zeros((L,), jnp.int32)
    ones16 = jnp.ones((L,), jnp.int32)
    cnt_ref[...] = zeros16
    base_ref[...] = zeros16
    tot_ref[...] = zeros16

    nvec = B // L
    before = wid * (CPT // L)
    for v in range(nvec):
        ev = idxall[pl.ds(v * L, L)]
        plsc.addupdate_scatter(tot_ref, [ev], ones16)
        in_before = jnp.full((L,), v < before)
        plsc.addupdate_scatter(base_ref, [ev], ones16, mask=in_before)

    pranks = []
    mvs = []
    for g in range(CPT // L):
        mv = idxall[pl.ds(tbase + g * L, L)]
        ofs = plsc.load_gather(cnt_ref, [mv])
        wr = zeros16
        for b in range(E):
            m = mv == b
            cs = plsc.cumsum(m.astype(jnp.int32))
            wr = jnp.where(m, cs - 1, wr)
        pranks.append(ofs + wr)
        mvs.append(mv)
        plsc.addupdate_scatter(cnt_ref, [mv], ones16)

    tot = tot_ref[...]
    nblk = (tot + (BLK - 1)) // BLK
    incl = plsc.cumsum(nblk)
    excl = incl - nblk
    bc_ref[...] = excl * BLK + base_ref[...]

    for g in range(CPT // L):
        pv = plsc.load_gather(bc_ref, [mvs[g]]) + pranks[g]
        posv[pl.ds(g * L, L)] = pv

    pltpu.sync_copy(posv, pos_hbm.at[pl.ds(tbase, CPT)])

    cp_x.wait()
    pltpu.async_copy(xrows, xs_hbm.at[posv], sem_sc).wait()

    @pl.when(wid == 0)
    def _():
        n_used = jnp.sum(nblk)
        lanes = jnp.arange(L, dtype=jnp.int32)
        mark_ref[pl.ds(0, L)] = zeros16
        mark_ref[pl.ds(L, L)] = zeros16
        plsc.store_scatter(mark_ref, [excl], lanes, mask=nblk > 0)
        cm0 = plsc.cummax(mark_ref[pl.ds(0, L)])
        carry = jnp.full((L,), jnp.max(cm0), jnp.int32)
        cm1 = jnp.maximum(plsc.cummax(mark_ref[pl.ds(L, L)]), carry)
        bexp_v[pl.ds(0, L)] = cm0
        bexp_v[pl.ds(L, L)] = cm1
        bexp_v[pl.ds(MAXB, L)] = jnp.full((L,), n_used, jnp.int32)
        pltpu.sync_copy(bexp_v, bexp_hbm)


def _route(x, idx_flat):
    mesh = plsc.VectorSubcoreMesh(core_axis_name="c", subcore_axis_name="s")
    f = pl.kernel(
        _route_body,
        out_type=[
            jax.ShapeDtypeStruct((NPAD, D), jnp.float32),
            jax.ShapeDtypeStruct((B,), jnp.int32),
            jax.ShapeDtypeStruct((MAXB + L,), jnp.int32),
        ],
        mesh=mesh,
        scratch_types=[
            pltpu.VMEM((CPT, D), jnp.float32),
            pltpu.VMEM((B,), jnp.int32),
            pltpu.VMEM((E,), jnp.int32),
            pltpu.VMEM((E,), jnp.int32),
            pltpu.VMEM((E,), jnp.int32),
            pltpu.VMEM((E,), jnp.int32),
            pltpu.VMEM((MAXB,), jnp.int32),
            pltpu.VMEM((CPT,), jnp.int32),
            pltpu.VMEM((MAXB + L,), jnp.int32),
            pltpu.SemaphoreType.DMA,
            pltpu.SemaphoreType.DMA,
        ],
        compiler_params=pltpu.CompilerParams(needs_layout_passes=False),
    )
    return f(x, idx_flat)


def _mlp_body(be_ref, xs_ref, w1_ref, b1_ref, w2_ref, b2_ref, ys_ref):
    j = pl.program_id(0)

    @pl.when(j < be_ref[NUSED_LANE])
    def _():
        e = be_ref[j]
        h = jnp.dot(xs_ref[...], w1_ref[0], preferred_element_type=jnp.float32)
        h = jnp.maximum(h + b1_ref[pl.ds(e, 1), :], 0.0)
        ys_ref[...] = (
            jnp.dot(h, w2_ref[0], preferred_element_type=jnp.float32)
            + b2_ref[pl.ds(e, 1), :]
        )


def _grouped_mlp(bexp, xs, W1, b1, W2, b2):
    def rowblk(j, be):
        return (jnp.minimum(j, be[NUSED_LANE] - 1), 0)

    grid_spec = pltpu.PrefetchScalarGridSpec(
        num_scalar_prefetch=1,
        grid=(MAXB,),
        in_specs=[
            pl.BlockSpec((BLK, D), rowblk),
            pl.BlockSpec((1, D, H), lambda j, be: (be[j], 0, 0)),
            pl.BlockSpec((E, H), lambda j, be: (0, 0)),
            pl.BlockSpec((1, H, D), lambda j, be: (be[j], 0, 0)),
            pl.BlockSpec((E, D), lambda j, be: (0, 0)),
        ],
        out_specs=pl.BlockSpec((BLK, D), rowblk),
    )
    return pl.pallas_call(
        _mlp_body,
        grid_spec=grid_spec,
        out_shape=jax.ShapeDtypeStruct((NPAD, D), jnp.float32),
    )(bexp, xs, W1, b1, W2, b2)


def _unsort_body(ys_hbm, pos_hbm, out_hbm, posv, rows, sem_g):
    c = lax.axis_index("c")
    s = lax.axis_index("s")
    wid = s * NC + c
    tbase = wid * CPT
    pltpu.sync_copy(pos_hbm.at[pl.ds(tbase, CPT)], posv)
    pltpu.async_copy(ys_hbm.at[posv], rows, sem_g).wait()
    pltpu.sync_copy(rows, out_hbm.at[pl.ds(tbase, CPT)])


def _unsort(ys, pos):
    mesh = plsc.VectorSubcoreMesh(core_axis_name="c", subcore_axis_name="s")
    f = pl.kernel(
        _unsort_body,
        out_type=jax.ShapeDtypeStruct((B, D), jnp.float32),
        mesh=mesh,
        scratch_types=[
            pltpu.VMEM((CPT,), jnp.int32),
            pltpu.VMEM((CPT, D), jnp.float32),
            pltpu.SemaphoreType.DMA,
        ],
        compiler_params=pltpu.CompilerParams(needs_layout_passes=False),
    )
    return f(ys, pos)


def kernel(x, Wg, bg, W1, b1, W2, b2, bias):
    logits, idx, idx1 = _gating(x, Wg, bg, bias)
    xs, pos, bexp = _route(x, idx1)
    ys = _grouped_mlp(bexp, xs, W1, b1, W2, b2)
    out = _unsort(ys, pos)
    return (out, logits, idx)
